# Initial kernel scaffold; baseline (speedup 1.0000x reference)
import jax
import jax.numpy as jnp
from jax.experimental import pallas as pl


def kernel(r, h, u, v, radius, latents):
    raise NotImplementedError("write your pallas kernel here")


# speedup vs baseline: 2.2362x; 2.2362x over previous
import functools

import jax
import jax.numpy as jnp
from jax import lax
from jax.experimental import pallas as pl
from jax.experimental.pallas import tpu as pltpu
from jax.experimental.pallas import tpu_sc as plsc

HX, HY, U, V, L = 16, 16, 128, 128, 16
LANES = 16
B = 128


def _sc_interp(h2, u, v, table, n_queries, num_workers):
    qpw = n_queries // num_workers
    nch = qpw // B
    mesh = plsc.VectorSubcoreMesh(core_axis_name="c", subcore_axis_name="s")
    nc = mesh.num_cores

    @functools.partial(
        pl.kernel,
        out_type=jax.ShapeDtypeStruct((n_queries, L), jnp.float32),
        mesh=mesh,
        compiler_params=pltpu.CompilerParams(
            needs_layout_passes=False, use_tc_tiling_on_sc=False),
        scratch_types=[
            pltpu.VMEM((2 * B,), jnp.float32),
            pltpu.VMEM((B,), jnp.float32),
            pltpu.VMEM((B,), jnp.float32),
            [pltpu.VMEM((B,), jnp.int32) for _ in range(4)],
            [pltpu.VMEM((B,), jnp.float32) for _ in range(4)],
            [pltpu.VMEM((B, L), jnp.float32) for _ in range(4)],
            pltpu.VMEM((B, L), jnp.float32),
            pltpu.SemaphoreType.DMA,
        ],
    )
    def k(h_hbm, u_hbm, v_hbm, tab_hbm, out_hbm,
          h_v, u_v, v_v, idx_vs, w_vs, row_vs, out_v, sem):
        wid = lax.axis_index("s") * nc + lax.axis_index("c")
        qbase0 = wid * qpw

        def chunk(c, carry):
            qb = qbase0 + c * B
            pltpu.sync_copy(h_hbm.at[pl.ds(qb * 2, 2 * B)], h_v)
            pltpu.sync_copy(u_hbm.at[pl.ds(qb, B)], u_v)
            pltpu.sync_copy(v_hbm.at[pl.ds(qb, B)], v_v)

            def sub(i, carry2):
                iota = lax.iota(jnp.int32, LANES)
                off = i * LANES
                q = iota + off
                h0 = plsc.load_gather(h_v, [q * 2])
                h1 = plsc.load_gather(h_v, [q * 2 + 1])
                uu = plsc.load_gather(u_v, [q])
                vv = plsc.load_gather(v_v, [q])
                ind_hx = (h0 + 1.0) / 2 * HX
                ind_hy = (h1 + 1.0) / 2 * HY
                ind_hx = jnp.where(ind_hx == float(HX), float(HX - 1), ind_hx)
                ind_hy = jnp.where(ind_hy == float(HY), float(HY - 1), ind_hy)
                i1 = ind_hx.astype(jnp.int32)
                j1 = ind_hy.astype(jnp.int32)
                ir = ind_hx - i1.astype(jnp.float32)
                jr = ind_hy - j1.astype(jnp.float32)
                i2 = lax.rem(i1 + 1, HX)
                j2 = lax.rem(j1 + 1, HY)
                ind_u = uu * U
                ind_v = vv * V
                ind_u = jnp.where(ind_u == float(U), float(U - 1), ind_u)
                ind_v = jnp.where(ind_v == float(V), float(V - 1), ind_v)
                ku = ind_u.astype(jnp.int32)
                kv = ind_v.astype(jnp.int32)
                buv = ku * V + kv
                stride = U * V
                plsc.store_scatter(idx_vs[0], [q], (i1 * HY + j1) * stride + buv)
                plsc.store_scatter(idx_vs[1], [q], (i1 * HY + j2) * stride + buv)
                plsc.store_scatter(idx_vs[2], [q], (i2 * HY + j1) * stride + buv)
                plsc.store_scatter(idx_vs[3], [q], (i2 * HY + j2) * stride + buv)
                omi = 1.0 - ir
                omj = 1.0 - jr
                plsc.store_scatter(w_vs[0], [q], omi * omj)
                plsc.store_scatter(w_vs[1], [q], omi * jr)
                plsc.store_scatter(w_vs[2], [q], ir * omj)
                plsc.store_scatter(w_vs[3], [q], ir * jr)
                return carry2

            lax.fori_loop(0, B // LANES, sub, 0)

            cps = [pltpu.async_copy(tab_hbm.at[idx_vs[t]], row_vs[t], sem)
                   for t in range(4)]
            for cp in cps:
                cp.wait()

            def grp(g, carry2):
                iota = lax.iota(jnp.int32, LANES)
                qv = iota + g * LANES
                wg = [plsc.load_gather(w_vs[t], [qv]) for t in range(4)]
                for d in range(L):
                    dv = jnp.full((LANES,), d, jnp.int32)
                    acc = wg[0] * plsc.load_gather(row_vs[0], [qv, dv])
                    acc += wg[1] * plsc.load_gather(row_vs[1], [qv, dv])
                    acc += wg[2] * plsc.load_gather(row_vs[2], [qv, dv])
                    acc += wg[3] * plsc.load_gather(row_vs[3], [qv, dv])
                    plsc.store_scatter(out_v, [qv, dv], acc)
                return carry2

            lax.fori_loop(0, B // LANES, grp, 0)
            pltpu.sync_copy(out_v, out_hbm.at[pl.ds(qb, B), :])
            return carry

        lax.fori_loop(0, nch, chunk, 0)

    return k(h2, u, v, table)


def kernel(r, h, u, v, radius, latents):
    del r, radius
    n = h.shape[0]
    info = plsc.get_sparse_core_info()
    num_workers = info.num_cores * info.num_subcores
    table = latents.reshape(HX * HY * U * V, L)
    h2 = h.reshape(-1)
    return _sc_interp(h2, u, v, table, n, num_workers)


# speedup vs baseline: 2.6196x; 1.1715x over previous
import functools

import jax
import jax.numpy as jnp
from jax import lax
from jax.experimental import pallas as pl
from jax.experimental.pallas import tpu as pltpu
from jax.experimental.pallas import tpu_sc as plsc

HX, HY, U, V, L = 16, 16, 128, 128, 16
LANES = 16
B = 128


def _sc_interp(h2, u, v, table, n_queries, num_workers):
    qpw = n_queries // num_workers
    nch = qpw // B
    mesh = plsc.VectorSubcoreMesh(core_axis_name="c", subcore_axis_name="s")
    nc = mesh.num_cores

    @functools.partial(
        pl.kernel,
        out_type=jax.ShapeDtypeStruct((n_queries, L), jnp.float32),
        mesh=mesh,
        compiler_params=pltpu.CompilerParams(
            needs_layout_passes=False, use_tc_tiling_on_sc=False),
        scratch_types=[
            [pltpu.VMEM((2 * B,), jnp.float32) for _ in range(2)],
            [pltpu.VMEM((B,), jnp.float32) for _ in range(2)],
            [pltpu.VMEM((B,), jnp.float32) for _ in range(2)],
            [[pltpu.VMEM((B,), jnp.int32) for _ in range(4)]
             for _ in range(2)],
            [[pltpu.VMEM((B,), jnp.float32) for _ in range(4)]
             for _ in range(2)],
            [[pltpu.VMEM((B, L), jnp.float32) for _ in range(4)]
             for _ in range(2)],
            [pltpu.VMEM((B, L), jnp.float32) for _ in range(2)],
            [pltpu.SemaphoreType.DMA for _ in range(2)],
            [pltpu.SemaphoreType.DMA for _ in range(2)],
            [pltpu.SemaphoreType.DMA for _ in range(2)],
        ],
    )
    def k(h_hbm, u_hbm, v_hbm, tab_hbm, out_hbm,
          h_vs, u_vs, v_vs, idx_vs, w_vs, row_vs, out_vs,
          in_sems, g_sems, out_sems):
        wid = lax.axis_index("s") * nc + lax.axis_index("c")
        qbase0 = wid * qpw
        iota = lax.iota(jnp.int32, LANES)

        def in_copies(c, p):
            qb = qbase0 + c * B
            return [
                (h_hbm.at[pl.ds(qb * 2, 2 * B)], h_vs[p]),
                (u_hbm.at[pl.ds(qb, B)], u_vs[p]),
                (v_hbm.at[pl.ds(qb, B)], v_vs[p]),
            ]

        def fire_in(c, p):
            for s, d in in_copies(c, p):
                pltpu.async_copy(s, d, in_sems[p])

        def wait_in(c, p):
            for s, d in in_copies(c, p):
                pltpu.make_async_copy(s, d, in_sems[p]).wait()

        def compute(c, p):
            def sub(i, carry):
                q = iota + i * LANES
                h0 = plsc.load_gather(h_vs[p], [q * 2])
                h1 = plsc.load_gather(h_vs[p], [q * 2 + 1])
                uu = plsc.load_gather(u_vs[p], [q])
                vv = plsc.load_gather(v_vs[p], [q])
                ind_hx = (h0 + 1.0) / 2 * HX
                ind_hy = (h1 + 1.0) / 2 * HY
                ind_hx = jnp.where(ind_hx == float(HX), float(HX - 1), ind_hx)
                ind_hy = jnp.where(ind_hy == float(HY), float(HY - 1), ind_hy)
                i1 = ind_hx.astype(jnp.int32)
                j1 = ind_hy.astype(jnp.int32)
                ir = ind_hx - i1.astype(jnp.float32)
                jr = ind_hy - j1.astype(jnp.float32)
                i2 = lax.rem(i1 + 1, HX)
                j2 = lax.rem(j1 + 1, HY)
                ind_u = uu * U
                ind_v = vv * V
                ind_u = jnp.where(ind_u == float(U), float(U - 1), ind_u)
                ind_v = jnp.where(ind_v == float(V), float(V - 1), ind_v)
                buv = ind_u.astype(jnp.int32) * V + ind_v.astype(jnp.int32)
                stride = U * V
                plsc.store_scatter(idx_vs[p][0], [q],
                                   (i1 * HY + j1) * stride + buv)
                plsc.store_scatter(idx_vs[p][1], [q],
                                   (i1 * HY + j2) * stride + buv)
                plsc.store_scatter(idx_vs[p][2], [q],
                                   (i2 * HY + j1) * stride + buv)
                plsc.store_scatter(idx_vs[p][3], [q],
                                   (i2 * HY + j2) * stride + buv)
                omi = 1.0 - ir
                omj = 1.0 - jr
                plsc.store_scatter(w_vs[p][0], [q], omi * omj)
                plsc.store_scatter(w_vs[p][1], [q], omi * jr)
                plsc.store_scatter(w_vs[p][2], [q], ir * omj)
                plsc.store_scatter(w_vs[p][3], [q], ir * jr)
                return carry

            lax.fori_loop(0, B // LANES, sub, 0)
            for t in range(4):
                pltpu.async_copy(tab_hbm.at[idx_vs[p][t]], row_vs[p][t],
                                 g_sems[p])

        def combine(c, p):
            @pl.when(c >= 2)
            def _():
                qb2 = qbase0 + (c - 2) * B
                pltpu.make_async_copy(
                    out_vs[p], out_hbm.at[pl.ds(qb2, B), :],
                    out_sems[p]).wait()

            for t in range(4):
                pltpu.make_async_copy(tab_hbm.at[idx_vs[p][t]], row_vs[p][t],
                                      g_sems[p]).wait()

            def grp(g, carry):
                qv = iota + g * LANES
                wg = [plsc.load_gather(w_vs[p][t], [qv]) for t in range(4)]
                for d in range(L):
                    dv = jnp.full((LANES,), d, jnp.int32)
                    acc = wg[0] * plsc.load_gather(row_vs[p][0], [qv, dv])
                    acc += wg[1] * plsc.load_gather(row_vs[p][1], [qv, dv])
                    acc += wg[2] * plsc.load_gather(row_vs[p][2], [qv, dv])
                    acc += wg[3] * plsc.load_gather(row_vs[p][3], [qv, dv])
                    plsc.store_scatter(out_vs[p], [qv, dv], acc)
                return carry

            lax.fori_loop(0, B // LANES, grp, 0)
            pltpu.async_copy(out_vs[p], out_hbm.at[pl.ds(qbase0 + c * B, B), :],
                             out_sems[p])

        def body(uu, carry):
            a = 2 * uu
            b = a + 1
            wait_in(a, 0)
            fire_in(b, 1)
            compute(a, 0)

            @pl.when(uu > 0)
            def _():
                combine(a - 1, 1)

            wait_in(b, 1)

            @pl.when(uu < nch // 2 - 1)
            def _():
                fire_in(a + 2, 0)

            compute(b, 1)
            combine(a, 0)
            return carry

        fire_in(0, 0)
        lax.fori_loop(0, nch // 2, body, 0)
        combine(nch - 1, 1)
        for p, c in ((0, nch - 2), (1, nch - 1)):
            pltpu.make_async_copy(
                out_vs[p], out_hbm.at[pl.ds(qbase0 + c * B, B), :],
                out_sems[p]).wait()

    return k(h2, u, v, table)


def kernel(r, h, u, v, radius, latents):
    del r, radius
    n = h.shape[0]
    info = plsc.get_sparse_core_info()
    num_workers = info.num_cores * info.num_subcores
    table = latents.reshape(HX * HY * U * V, L)
    h2 = h.reshape(-1)
    return _sc_interp(h2, u, v, table, n, num_workers)


# speedup vs baseline: 3.7754x; 1.4412x over previous
import functools

import jax
import jax.numpy as jnp
from jax import lax
from jax.experimental import pallas as pl
from jax.experimental.pallas import tpu as pltpu
from jax.experimental.pallas import tpu_sc as plsc

HX, HY, U, V, L = 16, 16, 128, 128, 16
LANES = 16
B = 128
PREP_ROWS = 512


def _prep_body(h0_ref, h1_ref, u_ref, v_ref,
               i00, i01, i10, i11, w00, w01, w10, w11):
    h0 = h0_ref[...]
    h1 = h1_ref[...]
    uu = u_ref[...]
    vv = v_ref[...]
    ind_hx = (h0 + 1.0) / 2 * HX
    ind_hy = (h1 + 1.0) / 2 * HY
    ind_hx = jnp.where(ind_hx == float(HX), float(HX - 1), ind_hx)
    ind_hy = jnp.where(ind_hy == float(HY), float(HY - 1), ind_hy)
    i1 = ind_hx.astype(jnp.int32)
    j1 = ind_hy.astype(jnp.int32)
    ir = ind_hx - i1.astype(jnp.float32)
    jr = ind_hy - j1.astype(jnp.float32)
    i2 = lax.rem(i1 + 1, HX)
    j2 = lax.rem(j1 + 1, HY)
    ind_u = uu * U
    ind_v = vv * V
    ind_u = jnp.where(ind_u == float(U), float(U - 1), ind_u)
    ind_v = jnp.where(ind_v == float(V), float(V - 1), ind_v)
    buv = ind_u.astype(jnp.int32) * V + ind_v.astype(jnp.int32)
    stride = U * V
    i00[...] = (i1 * HY + j1) * stride + buv
    i01[...] = (i1 * HY + j2) * stride + buv
    i10[...] = (i2 * HY + j1) * stride + buv
    i11[...] = (i2 * HY + j2) * stride + buv
    omi = 1.0 - ir
    omj = 1.0 - jr
    w00[...] = omi * omj
    w01[...] = omi * jr
    w10[...] = ir * omj
    w11[...] = ir * jr


def _tc_prep(h0, h1, u, v):
    rows = h0.shape[0]
    blk = pl.BlockSpec((PREP_ROWS, 128), lambda g: (g, 0))
    f32 = jnp.float32
    return pl.pallas_call(
        _prep_body,
        grid=(rows // PREP_ROWS,),
        in_specs=[blk, blk, blk, blk],
        out_specs=[blk] * 8,
        out_shape=[jax.ShapeDtypeStruct((rows, 128), jnp.int32)] * 4
        + [jax.ShapeDtypeStruct((rows, 128), f32)] * 4,
    )(h0, h1, u, v)


def _sc_interp(idxs, ws, table, n_queries, num_workers):
    qpw = n_queries // num_workers
    nch = qpw // B
    out_rows = n_queries * L // 128
    mesh = plsc.VectorSubcoreMesh(core_axis_name="c", subcore_axis_name="s")
    nc = mesh.num_cores

    @functools.partial(
        pl.kernel,
        out_type=jax.ShapeDtypeStruct((out_rows, 128), jnp.float32),
        mesh=mesh,
        compiler_params=pltpu.CompilerParams(
            needs_layout_passes=False, use_tc_tiling_on_sc=False),
        scratch_types=[
            [[pltpu.VMEM((B,), jnp.int32) for _ in range(4)]
             for _ in range(4)],
            [[pltpu.VMEM((B,), jnp.float32) for _ in range(4)]
             for _ in range(4)],
            [[pltpu.VMEM((B, L), jnp.float32) for _ in range(4)]
             for _ in range(4)],
            [pltpu.VMEM((B * L // 128, 128), jnp.float32)
             for _ in range(4)],
            [pltpu.SemaphoreType.DMA for _ in range(4)],
            [pltpu.SemaphoreType.DMA for _ in range(4)],
            [pltpu.SemaphoreType.DMA for _ in range(4)],
        ],
    )
    def k(i0_hbm, i1_hbm, i2_hbm, i3_hbm, w0_hbm, w1_hbm, w2_hbm, w3_hbm,
          tab_hbm, out_hbm,
          idx_vs, w_vs, row_vs, out_vs, in_sems, g_sems, out_sems):
        i_hbms = (i0_hbm, i1_hbm, i2_hbm, i3_hbm)
        w_hbms = (w0_hbm, w1_hbm, w2_hbm, w3_hbm)
        wid = lax.axis_index("s") * nc + lax.axis_index("c")
        qbase0 = wid * qpw
        iota = lax.iota(jnp.int32, LANES)
        orows = B * L // 128

        def in_copies(c, p):
            qb = qbase0 + c * B
            cps = []
            for t in range(4):
                cps.append((i_hbms[t].at[pl.ds(qb, B)], idx_vs[p][t]))
                cps.append((w_hbms[t].at[pl.ds(qb, B)], w_vs[p][t]))
            return cps

        def fire_in(c, p):
            for s, d in in_copies(c, p):
                pltpu.async_copy(s, d, in_sems[p])

        def wait_in(c, p):
            for s, d in in_copies(c, p):
                pltpu.make_async_copy(s, d, in_sems[p]).wait()

        def fire_gathers(p):
            for t in range(4):
                pltpu.async_copy(tab_hbm.at[idx_vs[p][t]], row_vs[p][t],
                                 g_sems[p])

        def out_slice(c):
            return out_hbm.at[pl.ds((qbase0 + c * B) * L // 128, orows), :]

        def combine(c, p):
            @pl.when(c >= 4)
            def _():
                pltpu.make_async_copy(out_vs[p], out_slice(c - 4),
                                      out_sems[p]).wait()

            for t in range(4):
                pltpu.make_async_copy(tab_hbm.at[idx_vs[p][t]], row_vs[p][t],
                                      g_sems[p]).wait()

            def grp(g, carry):
                qv = iota + g * LANES
                orow = lax.shift_right_logical(qv, 3)
                olane0 = lax.shift_left(lax.bitwise_and(qv, 7), 4)
                wg = [plsc.load_gather(w_vs[p][t], [qv]) for t in range(4)]
                for d in range(L):
                    dv = jnp.full((LANES,), d, jnp.int32)
                    acc = wg[0] * plsc.load_gather(row_vs[p][0], [qv, dv])
                    acc += wg[1] * plsc.load_gather(row_vs[p][1], [qv, dv])
                    acc += wg[2] * plsc.load_gather(row_vs[p][2], [qv, dv])
                    acc += wg[3] * plsc.load_gather(row_vs[p][3], [qv, dv])
                    plsc.store_scatter(out_vs[p], [orow, olane0 + d], acc)
                return carry

            lax.fori_loop(0, B // LANES, grp, 0)
            pltpu.async_copy(out_vs[p], out_slice(c), out_sems[p])

        def body(uu, carry):
            c0 = 4 * uu

            def step(q, pin):
                c = c0 + q
                wait_in(c, q)
                fire_gathers(q)

                @pl.when(c - 1 >= 0)
                def _():
                    combine(c - 1, (q - 1) % 4)

                @pl.when(pin < nch)
                def _():
                    fire_in(pin, (q + 2) % 4)

            step(0, c0 + 2)
            step(1, c0 + 3)
            step(2, c0 + 4)
            step(3, c0 + 5)
            return carry

        fire_in(0, 0)
        fire_in(1, 1)
        lax.fori_loop(0, nch // 4, body, 0)
        combine(nch - 1, 3)
        for p in range(4):
            c = nch - 4 + p
            pltpu.make_async_copy(out_vs[p], out_slice(c), out_sems[p]).wait()

    return k(*idxs, *ws, table)


def kernel(r, h, u, v, radius, latents):
    del r, radius
    n = h.shape[0]
    info = plsc.get_sparse_core_info()
    num_workers = info.num_cores * info.num_subcores
    rows = n // 128
    h0 = h[:, 0].reshape(rows, 128)
    h1 = h[:, 1].reshape(rows, 128)
    ub = u.reshape(rows, 128)
    vb = v.reshape(rows, 128)
    prep = _tc_prep(h0, h1, ub, vb)
    idxs = [x.reshape(n) for x in prep[:4]]
    ws = [x.reshape(n) for x in prep[4:]]
    t128 = lax.optimization_barrier(latents.reshape(-1, 128))
    table = t128.reshape(HX * HY * U * V, L)
    out128 = _sc_interp(idxs, ws, table, n, num_workers)
    return out128.reshape(n, L)
